# baseline (speedup 1.0000x reference)
import functools

import jax
import jax.numpy as jnp
import numpy as np
from jax import lax
from jax.experimental import pallas as pl
from jax.experimental.pallas import tpu as pltpu
from jax.experimental.pallas import tpu_sc as plsc

N = 10000
E = 320000
F = 128
R = 16
R4F = 512
NT = 4

NC, NS, L = 2, 16, 16
CH = 80
NROWS_ALL = E // CH
RPT = NROWS_ALL // NS
NBK = 10
NBLK = RPT // NBK
NROW = 640
N_PAD = NROW * NS
ZR = 20
NROW_LAST = N - NROW * (NS - 1)

BN = 2000
BE = 4000

_PERM = np.empty((F,), dtype=np.int32)
for _b in range(F // 32):
    for _i in range(16):
        _PERM[32 * _b + 2 * _i] = 32 * _b + _i
        _PERM[32 * _b + 2 * _i + 1] = 32 * _b + 16 + _i
_PERM4F = np.concatenate([_PERM + 128 * _t for _t in range(NT)])


def _node_tc_body(s_ref, v_ref, w1t_ref, b1_ref, w2t_ref, b2_ref, g_ref):
    h = jnp.dot(s_ref[...], w1t_ref[...], preferred_element_type=jnp.float32)
    h = h + b1_ref[...]
    h = h * jax.nn.sigmoid(h)
    phi = jnp.dot(h, w2t_ref[...], preferred_element_type=jnp.float32) + b2_ref[...]
    g_ref[0] = phi[:, :F]
    g_ref[1] = phi[:, F:2 * F] * v_ref[...]
    g_ref[2] = phi[:, 2 * F:3 * F]
    g_ref[3] = phi[:, 3 * F:4 * F]


def _edge_tc_body(r1_ref, r2_ref, fc1_ref, fc2_ref, u1_ref, u2_ref, wrt_ref,
                  br_ref, w_ref):
    fc1 = fc1_ref[...]
    fc2 = fc2_ref[...]
    a = fc1 * r1_ref[...] + fc2 * r2_ref[...]
    w = jnp.dot(a, wrt_ref[...], preferred_element_type=jnp.float32)
    w = w + (fc1 + fc2) * br_ref[...]
    w_ref[0] = w[:, :F].astype(jnp.bfloat16)
    w_ref[1] = w[:, F:2 * F].astype(jnp.bfloat16)
    w_ref[2] = (w[:, 2 * F:3 * F] * u1_ref[...]).astype(jnp.bfloat16)
    w_ref[3] = (w[:, 3 * F:4 * F] * u2_ref[...]).astype(jnp.bfloat16)


def _combine_tc_body(s_ref, v_ref, p_ref, os_ref, ov_ref):
    os_ref[...] = s_ref[...] + p_ref[0]
    ov_ref[...] = v_ref[...] + (p_ref[1] + p_ref[2] + p_ref[3])


_sc_mesh = plsc.VectorSubcoreMesh(core_axis_name="c", subcore_axis_name="s")


@functools.partial(
    pl.kernel,
    out_type=jax.ShapeDtypeStruct((NT, N, F), jnp.float32),
    mesh=_sc_mesh,
    scratch_types=[
        pltpu.VMEM((NBK, CH), jnp.int32),
        pltpu.VMEM((NBK, CH), jnp.int32),
        pltpu.VMEM((CH, F), jnp.float32),
        pltpu.VMEM((CH, F), jnp.float32),
        pltpu.VMEM((CH // 2, F), jnp.int32),
        pltpu.VMEM((CH // 2, F), jnp.int32),
        pltpu.VMEM((CH, F), jnp.float32),
        pltpu.VMEM((ZR, F), jnp.float32),
        pltpu.VMEM_SHARED((N_PAD, F), jnp.float32),
        pltpu.SemaphoreType.DMA,
        pltpu.SemaphoreType.DMA,
        pltpu.SemaphoreType.DMA,
        pltpu.SemaphoreType.DMA,
    ],
)
def _sc_scatter4(g_hbm, w_hbm, idxj_hbm, idxi_hbm, out_hbm,
                 idxj_blk, idxi_blk, g_a, g_b, w_a, w_b, m_v, z_v, acc,
                 sem_ga, sem_gb, sem_wa, sem_wb):
    cid = lax.axis_index("c")
    sid = lax.axis_index("s")
    nbase = pl.multiple_of(sid * NROW, 8)

    zero = jnp.zeros((L,), jnp.float32)

    def zrow(rr, carry):
        for k in range(F // L):
            z_v[rr, pl.ds(k * L, L)] = zero
        return carry

    lax.fori_loop(0, ZR, zrow, 0)

    def zero_acc():
        for q in range(NROW // ZR):
            pltpu.sync_copy(z_v, acc.at[pl.ds(nbase + q * ZR, ZR)])

    himask = jnp.full((L,), -65536, jnp.int32)

    def compute(g_v, w_v):
        def edge(m, icarry):
            for j in range(2):
                c = 2 * m + j
                for k in range(F // 32):
                    wi = w_v[m, pl.ds(j * 64 + k * L, L)]
                    wlo = lax.bitcast_convert_type(
                        lax.shift_left(wi, 16), jnp.float32)
                    whi = lax.bitcast_convert_type(wi & himask, jnp.float32)
                    m_v[c, pl.ds(k * 32, L)] = g_v[c, pl.ds(k * 32, L)] * wlo
                    m_v[c, pl.ds(k * 32 + L, L)] = (
                        g_v[c, pl.ds(k * 32 + L, L)] * whi)
            return icarry

        lax.fori_loop(0, CH // 2, edge, 0)

    def run_task(tid):
        rbase = sid * RPT

        def issue(row, rid, g_v, w_v, sem_g, sem_w):
            pltpu.async_copy(g_hbm.at[idxj_blk.at[row]], g_v, sem_g)
            pltpu.async_copy(w_hbm.at[rid], w_v, sem_w)

        def wait(g_v, w_v, sem_g, sem_w):
            pltpu.make_async_copy(g_hbm.at[pl.ds(0, CH)], g_v, sem_g).wait()
            pltpu.make_async_copy(w_hbm.at[0], w_v, sem_w).wait()

        def block(blk, carry):
            q0 = rbase + blk * NBK
            pltpu.sync_copy(idxj_hbm.at[tid, sid, blk], idxj_blk)
            pltpu.sync_copy(idxi_hbm.at[sid, blk], idxi_blk)
            rid0 = tid * NROWS_ALL + q0
            issue(0, rid0, g_a, w_a, sem_ga, sem_wa)

            def pair(k, icarry):
                r0 = 2 * k
                issue(r0 + 1, rid0 + r0 + 1, g_b, w_b, sem_gb, sem_wb)
                wait(g_a, w_a, sem_ga, sem_wa)
                compute(g_a, w_a)
                pltpu.sync_copy(m_v, acc.at[idxi_blk.at[r0]], add=True)

                @pl.when(k < NBK // 2 - 1)
                def _prefetch():
                    issue(r0 + 2, rid0 + r0 + 2, g_a, w_a, sem_ga, sem_wa)

                wait(g_b, w_b, sem_gb, sem_wb)
                compute(g_b, w_b)
                pltpu.sync_copy(m_v, acc.at[idxi_blk.at[r0 + 1]], add=True)
                return icarry

            lax.fori_loop(0, NBK // 2, pair, 0)
            return carry

        lax.fori_loop(0, NBLK, block, 0)

    def copy_out(tid):
        @pl.when(sid != NS - 1)
        def _copy_full():
            pltpu.sync_copy(acc.at[pl.ds(nbase, NROW)],
                            out_hbm.at[tid, pl.ds(nbase, NROW)])

        @pl.when(sid == NS - 1)
        def _copy_tail():
            pltpu.sync_copy(acc.at[pl.ds(nbase, NROW_LAST)],
                            out_hbm.at[tid, pl.ds(nbase, NROW_LAST)])

    for q in range(NT // NC):
        tid = cid * (NT // NC) + q
        zero_acc()
        plsc.subcore_barrier()
        run_task(tid)
        plsc.subcore_barrier()
        copy_out(tid)


def kernel(s, v, radial_embeddings_1, radial_embeddings_2, f_cut_1, f_cut_2,
           unit_vectors_1, unit_vectors_2, edge_index, W1, b1, W2, b2, Wr, br):
    idx_i = edge_index[0].astype(jnp.int32)
    idx_j = edge_index[1].astype(jnp.int32)
    idxi2d = idx_i.reshape(NS, NBLK, NBK, CH)
    idxj4 = (idx_j.reshape(NROWS_ALL, CH)[None]
             + (jnp.arange(NT, dtype=jnp.int32) * N)[:, None, None]
             ).reshape(NT, NS, NBLK, NBK, CH)
    fc1 = f_cut_1.reshape(E, 1)
    fc2 = f_cut_2.reshape(E, 1)
    u1 = unit_vectors_1.reshape(E, 1)
    u2 = unit_vectors_2.reshape(E, 1)
    wrt_p = Wr.T[:, _PERM4F]
    br_p = br[_PERM4F].reshape(1, R4F)

    g4 = pl.pallas_call(
        _node_tc_body,
        grid=(N // BN,),
        in_specs=[
            pl.BlockSpec((BN, F), lambda i: (i, 0)),
            pl.BlockSpec((BN, F), lambda i: (i, 0)),
            pl.BlockSpec((F, F), lambda i: (0, 0)),
            pl.BlockSpec((1, F), lambda i: (0, 0)),
            pl.BlockSpec((F, R4F), lambda i: (0, 0)),
            pl.BlockSpec((1, R4F), lambda i: (0, 0)),
        ],
        out_specs=pl.BlockSpec((NT, BN, F), lambda i: (0, i, 0)),
        out_shape=jax.ShapeDtypeStruct((NT, N, F), jnp.float32),
    )(s, v, W1.T, b1.reshape(1, F), W2.T, b2.reshape(1, R4F))

    w4 = pl.pallas_call(
        _edge_tc_body,
        grid=(E // BE,),
        in_specs=[
            pl.BlockSpec((BE, R), lambda i: (i, 0)),
            pl.BlockSpec((BE, R), lambda i: (i, 0)),
            pl.BlockSpec((BE, 1), lambda i: (i, 0)),
            pl.BlockSpec((BE, 1), lambda i: (i, 0)),
            pl.BlockSpec((BE, 1), lambda i: (i, 0)),
            pl.BlockSpec((BE, 1), lambda i: (i, 0)),
            pl.BlockSpec((R, R4F), lambda i: (0, 0)),
            pl.BlockSpec((1, R4F), lambda i: (0, 0)),
        ],
        out_specs=pl.BlockSpec((NT, BE, F), lambda i: (0, i, 0)),
        out_shape=jax.ShapeDtypeStruct((NT, E, F), jnp.bfloat16),
    )(radial_embeddings_1, radial_embeddings_2, fc1, fc2, u1, u2,
      wrt_p, br_p)

    w4i = jax.lax.bitcast_convert_type(
        w4.reshape(NT * NROWS_ALL, CH // 2, F, 2), jnp.int32)
    p4 = _sc_scatter4(g4.reshape(NT * N, F), w4i, idxj4, idxi2d)

    out_s, out_v = pl.pallas_call(
        _combine_tc_body,
        grid=(N // BN,),
        in_specs=[
            pl.BlockSpec((BN, F), lambda i: (i, 0)),
            pl.BlockSpec((BN, F), lambda i: (i, 0)),
            pl.BlockSpec((NT, BN, F), lambda i: (0, i, 0)),
        ],
        out_specs=[
            pl.BlockSpec((BN, F), lambda i: (i, 0)),
            pl.BlockSpec((BN, F), lambda i: (i, 0)),
        ],
        out_shape=[
            jax.ShapeDtypeStruct((N, F), jnp.float32),
            jax.ShapeDtypeStruct((N, F), jnp.float32),
        ],
    )(s, v, p4)

    return out_s, out_v


# speedup vs baseline: 28.0228x; 28.0228x over previous
import functools

import jax
import jax.numpy as jnp
from jax import lax
from jax.experimental import pallas as pl
from jax.experimental.pallas import tpu as pltpu
from jax.experimental.pallas import tpu_sc as plsc

N = 10000
E = 320000
F = 128
R = 16
R4F = 512
NT = 4

NC, NS, L = 2, 16, 16
CH = 40
NROWS_ALL = E // CH
RPT = NROWS_ALL // NS
NBK = 100
NBLK = RPT // NBK
QPB = NBK // 4
NROW = 640
N_PAD = NROW * NS
ZR = 20
NROW_LAST = N - NROW * (NS - 1)

BN = 2000
BE = 4000


def _node_tc_body(s_ref, v_ref, w1t_ref, b1_ref, w2t_ref, b2_ref, g_ref):
    t = pl.program_id(0)
    h = jnp.dot(s_ref[...], w1t_ref[...], preferred_element_type=jnp.float32)
    h = h + b1_ref[...]
    h = h * jax.nn.sigmoid(h)
    g = jnp.dot(h, w2t_ref[...], preferred_element_type=jnp.float32) + b2_ref[...]
    g_ref[...] = jnp.where(t == 1, g * v_ref[...], g)


def _edge_tc_body(r1_ref, r2_ref, fc1_ref, fc2_ref, u1_ref, u2_ref, wrt_ref,
                  br_ref, w_ref):
    fc1 = fc1_ref[...]
    fc2 = fc2_ref[...]
    a = fc1 * r1_ref[...] + fc2 * r2_ref[...]
    w = jnp.dot(a, wrt_ref[...], preferred_element_type=jnp.float32)
    w = w + (fc1 + fc2) * br_ref[...]
    w_ref[0] = w[:, :F]
    w_ref[1] = w[:, F:2 * F]
    w_ref[2] = w[:, 2 * F:3 * F] * u1_ref[...]
    w_ref[3] = w[:, 3 * F:4 * F] * u2_ref[...]


def _combine_tc_body(s_ref, v_ref, p_ref, os_ref, ov_ref):
    os_ref[...] = s_ref[...] + p_ref[0]
    ov_ref[...] = v_ref[...] + (p_ref[1] + p_ref[2] + p_ref[3])


_sc_mesh = plsc.VectorSubcoreMesh(core_axis_name="c", subcore_axis_name="s")

IB = NBK * CH


@functools.partial(
    pl.kernel,
    out_type=jax.ShapeDtypeStruct((NT, N, F), jnp.float32),
    mesh=_sc_mesh,
    scratch_types=[
        pltpu.VMEM((IB,), jnp.int32),
        [pltpu.VMEM((CH,), jnp.int32)] * 4,
        [pltpu.VMEM((CH, F), jnp.float32)] * 2,
        [pltpu.VMEM((CH, F), jnp.float32)] * 2,
        [pltpu.VMEM((CH, F), jnp.float32)] * 2,
        pltpu.VMEM((ZR, F), jnp.float32),
        pltpu.VMEM_SHARED((N_PAD, F), jnp.float32),
        [pltpu.SemaphoreType.DMA] * 2,
        [pltpu.SemaphoreType.DMA] * 2,
        [pltpu.SemaphoreType.DMA] * 2,
        [pltpu.SemaphoreType.DMA] * 4,
    ],
)
def _sc_scatter4(g_hbm, w_hbm, idxj_hbm, idxi_hbm, out_hbm,
                 idxj_blk, ri, g_b, w_b, m_b, z_v, acc,
                 sem_g, sem_w, sem_s, sem_i):
    cid = lax.axis_index("c")
    sid = lax.axis_index("s")
    nbase = pl.multiple_of(sid * NROW, 8)

    zero = jnp.zeros((L,), jnp.float32)

    def zrow(rr, carry):
        for k in range(F // L):
            z_v[rr, pl.ds(k * L, L)] = zero
        return carry

    lax.fori_loop(0, ZR, zrow, 0)

    def zero_acc():
        for q in range(NROW // ZR):
            pltpu.sync_copy(z_v, acc.at[pl.ds(nbase + q * ZR, ZR)])

    def compute(g_v, w_v, m_v):
        def edge(c, icarry):
            for k in range(F // L):
                m_v[c, pl.ds(k * L, L)] = (
                    g_v[c, pl.ds(k * L, L)] * w_v[c, pl.ds(k * L, L)])
            return icarry

        lax.fori_loop(0, CH, edge, 0)

    def run_task(tid):
        rbase = sid * RPT

        def issue(r, e_blk, p, slot):
            i0 = pl.multiple_of(r * CH, 8)
            pltpu.async_copy(g_hbm.at[idxj_blk.at[pl.ds(i0, CH)]],
                             g_b[p], sem_g[p])
            e0 = pl.multiple_of(e_blk + r * CH, 8)
            pltpu.async_copy(w_hbm.at[tid, pl.ds(e0, CH)], w_b[p], sem_w[p])
            pltpu.async_copy(idxi_hbm.at[pl.ds(e0, CH)], ri[slot], sem_i[slot])

        def wait_in(p):
            pltpu.make_async_copy(g_hbm.at[pl.ds(0, CH)], g_b[p], sem_g[p]).wait()
            pltpu.make_async_copy(w_hbm.at[0, pl.ds(0, CH)], w_b[p],
                                  sem_w[p]).wait()

        def wait_scatter(p):
            pltpu.make_async_copy(m_b[p], acc.at[pl.ds(0, CH)], sem_s[p]).wait()

        def block(blk, carry):
            e_blk = (rbase + blk * NBK) * CH
            pltpu.sync_copy(idxj_hbm.at[pl.ds(tid * E + e_blk, IB)], idxj_blk)
            issue(0, e_blk, 0, 0)

            def quad(q, icarry):
                for jj in range(4):
                    r = 4 * q + jj
                    p = jj % 2
                    if jj < 3:
                        issue(r + 1, e_blk, 1 - p, jj + 1)
                    else:
                        @pl.when(q < QPB - 1)
                        def _issue_next():
                            issue(r + 1, e_blk, 1 - p, 0)
                    wait_in(p)
                    if jj < 2:
                        @pl.when(q > 0)
                        def _ws():
                            wait_scatter(p)
                    else:
                        wait_scatter(p)
                    compute(g_b[p], w_b[p], m_b[p])
                    pltpu.make_async_copy(idxi_hbm.at[pl.ds(0, CH)], ri[jj],
                                          sem_i[jj]).wait()
                    pltpu.async_copy(m_b[p], acc.at[ri[jj]], sem_s[p], add=True)
                return icarry

            lax.fori_loop(0, QPB, quad, 0)
            wait_scatter(0)
            wait_scatter(1)
            return carry

        lax.fori_loop(0, NBLK, block, 0)

    def copy_out(tid):
        @pl.when(sid != NS - 1)
        def _copy_full():
            pltpu.sync_copy(acc.at[pl.ds(nbase, NROW)],
                            out_hbm.at[tid, pl.ds(nbase, NROW)])

        @pl.when(sid == NS - 1)
        def _copy_tail():
            pltpu.sync_copy(acc.at[pl.ds(nbase, NROW_LAST)],
                            out_hbm.at[tid, pl.ds(nbase, NROW_LAST)])

    for q in range(NT // NC):
        tid = cid * (NT // NC) + q
        zero_acc()
        plsc.subcore_barrier()
        run_task(tid)
        plsc.subcore_barrier()
        copy_out(tid)


def kernel(s, v, radial_embeddings_1, radial_embeddings_2, f_cut_1, f_cut_2,
           unit_vectors_1, unit_vectors_2, edge_index, W1, b1, W2, b2, Wr, br):
    idx_i = edge_index[0].astype(jnp.int32)
    idx_j = edge_index[1].astype(jnp.int32)
    idxj4 = (idx_j[None, :]
             + (jnp.arange(NT, dtype=jnp.int32) * N)[:, None]).reshape(NT * E)
    fc1 = f_cut_1.reshape(E, 1)
    fc2 = f_cut_2.reshape(E, 1)
    u1 = unit_vectors_1.reshape(E, 1)
    u2 = unit_vectors_2.reshape(E, 1)

    g4 = pl.pallas_call(
        _node_tc_body,
        grid=(NT, N // BN),
        in_specs=[
            pl.BlockSpec((BN, F), lambda t, i: (i, 0)),
            pl.BlockSpec((BN, F), lambda t, i: (i, 0)),
            pl.BlockSpec((F, F), lambda t, i: (0, 0)),
            pl.BlockSpec((1, F), lambda t, i: (0, 0)),
            pl.BlockSpec((F, F), lambda t, i: (0, t)),
            pl.BlockSpec((1, F), lambda t, i: (0, t)),
        ],
        out_specs=pl.BlockSpec((BN, F), lambda t, i: (t * (N // BN) + i, 0)),
        out_shape=jax.ShapeDtypeStruct((NT * N, F), jnp.float32),
    )(s, v, W1.T, b1.reshape(1, F), W2.T, b2.reshape(1, R4F))

    w4 = pl.pallas_call(
        _edge_tc_body,
        grid=(E // BE,),
        in_specs=[
            pl.BlockSpec((BE, R), lambda i: (i, 0)),
            pl.BlockSpec((BE, R), lambda i: (i, 0)),
            pl.BlockSpec((BE, 1), lambda i: (i, 0)),
            pl.BlockSpec((BE, 1), lambda i: (i, 0)),
            pl.BlockSpec((BE, 1), lambda i: (i, 0)),
            pl.BlockSpec((BE, 1), lambda i: (i, 0)),
            pl.BlockSpec((R, R4F), lambda i: (0, 0)),
            pl.BlockSpec((1, R4F), lambda i: (0, 0)),
        ],
        out_specs=pl.BlockSpec((NT, BE, F), lambda i: (0, i, 0)),
        out_shape=jax.ShapeDtypeStruct((NT, E, F), jnp.float32),
    )(radial_embeddings_1, radial_embeddings_2, fc1, fc2, u1, u2,
      Wr.T, br.reshape(1, R4F))

    p4 = _sc_scatter4(g4, w4, idxj4, idx_i)

    out_s, out_v = pl.pallas_call(
        _combine_tc_body,
        grid=(N // BN,),
        in_specs=[
            pl.BlockSpec((BN, F), lambda i: (i, 0)),
            pl.BlockSpec((BN, F), lambda i: (i, 0)),
            pl.BlockSpec((NT, BN, F), lambda i: (0, i, 0)),
        ],
        out_specs=[
            pl.BlockSpec((BN, F), lambda i: (i, 0)),
            pl.BlockSpec((BN, F), lambda i: (i, 0)),
        ],
        out_shape=[
            jax.ShapeDtypeStruct((N, F), jnp.float32),
            jax.ShapeDtypeStruct((N, F), jnp.float32),
        ],
    )(s, v, p4)

    return out_s, out_v


# speedup vs baseline: 28.2530x; 1.0082x over previous
import functools

import jax
import jax.numpy as jnp
from jax import lax
from jax.experimental import pallas as pl
from jax.experimental.pallas import tpu as pltpu
from jax.experimental.pallas import tpu_sc as plsc

N = 10000
E = 320000
F = 128
R = 16
R4F = 512
NT = 4

NC, NS, L = 2, 16, 16
CH = 40
NROWS_ALL = E // CH
RPT = NROWS_ALL // NS
NROW = 640
N_PAD = NROW * NS
ZR = 20
NROW_LAST = N - NROW * (NS - 1)

BN = 2000
BE = 4000


def _node_tc_body(s_ref, v_ref, w1t_ref, b1_ref, w2t_ref, b2_ref, g_ref):
    t = pl.program_id(0)
    h = jnp.dot(s_ref[...], w1t_ref[...], preferred_element_type=jnp.float32)
    h = h + b1_ref[...]
    h = h * jax.nn.sigmoid(h)
    g = jnp.dot(h, w2t_ref[...], preferred_element_type=jnp.float32) + b2_ref[...]
    g_ref[...] = jnp.where(t == 1, g * v_ref[...], g)


def _edge_tc_body(r1_ref, r2_ref, fc1_ref, fc2_ref, u1_ref, u2_ref, wrt_ref,
                  br_ref, w_ref):
    fc1 = fc1_ref[...]
    fc2 = fc2_ref[...]
    a = fc1 * r1_ref[...] + fc2 * r2_ref[...]
    w = jnp.dot(a, wrt_ref[...], preferred_element_type=jnp.float32)
    w = w + (fc1 + fc2) * br_ref[...]
    w_ref[0] = w[:, :F]
    w_ref[1] = w[:, F:2 * F]
    w_ref[2] = w[:, 2 * F:3 * F] * u1_ref[...]
    w_ref[3] = w[:, 3 * F:4 * F] * u2_ref[...]


def _combine_tc_body(s_ref, v_ref, p_ref, os_ref, ov_ref):
    os_ref[...] = s_ref[...] + p_ref[0]
    ov_ref[...] = v_ref[...] + (p_ref[1] + p_ref[2] + p_ref[3])


_sc_mesh = plsc.VectorSubcoreMesh(core_axis_name="c", subcore_axis_name="s")

@functools.partial(
    pl.kernel,
    out_type=jax.ShapeDtypeStruct((NT, N, F), jnp.float32),
    mesh=_sc_mesh,
    scratch_types=[
        [pltpu.VMEM((CH,), jnp.int32)] * 4,
        [pltpu.VMEM((CH,), jnp.int32)] * 4,
        [pltpu.VMEM((CH, F), jnp.float32)] * 2,
        [pltpu.VMEM((CH, F), jnp.float32)] * 2,
        [pltpu.VMEM((CH, F), jnp.float32)] * 2,
        pltpu.VMEM((ZR, F), jnp.float32),
        pltpu.VMEM_SHARED((N_PAD, F), jnp.float32),
        [pltpu.SemaphoreType.DMA] * 2,
        [pltpu.SemaphoreType.DMA] * 2,
        [pltpu.SemaphoreType.DMA] * 2,
        [pltpu.SemaphoreType.DMA] * 4,
        [pltpu.SemaphoreType.DMA] * 4,
    ],
)
def _sc_scatter4(g_hbm, w_hbm, idxj_hbm, idxi_hbm, out_hbm,
                 rj, ri, g_b, w_b, m_b, z_v, acc,
                 sem_g, sem_w, sem_s, sem_j, sem_i):
    cid = lax.axis_index("c")
    sid = lax.axis_index("s")
    nbase = pl.multiple_of(sid * NROW, 8)

    zero = jnp.zeros((L,), jnp.float32)

    def zrow(rr, carry):
        for k in range(F // L):
            z_v[rr, pl.ds(k * L, L)] = zero
        return carry

    lax.fori_loop(0, ZR, zrow, 0)

    def zero_acc():
        for q in range(NROW // ZR):
            pltpu.sync_copy(z_v, acc.at[pl.ds(nbase + q * ZR, ZR)])

    def compute(g_v, w_v, m_v):
        def edge(c, icarry):
            for k in range(F // L):
                m_v[c, pl.ds(k * L, L)] = (
                    g_v[c, pl.ds(k * L, L)] * w_v[c, pl.ds(k * L, L)])
            return icarry

        lax.fori_loop(0, CH, edge, 0)

    def run_task(tid):
        rbase = sid * RPT

        def issue_idx(r, sl):
            e0 = pl.multiple_of((rbase + r) * CH, 8)
            pltpu.async_copy(idxj_hbm.at[pl.ds(tid * E + e0, CH)],
                             rj[sl], sem_j[sl])
            pltpu.async_copy(idxi_hbm.at[pl.ds(e0, CH)], ri[sl], sem_i[sl])

        def wait_j(sl):
            pltpu.make_async_copy(idxi_hbm.at[pl.ds(0, CH)], rj[sl],
                                  sem_j[sl]).wait()

        def wait_i(sl):
            pltpu.make_async_copy(idxi_hbm.at[pl.ds(0, CH)], ri[sl],
                                  sem_i[sl]).wait()

        def issue_gw(r, p, sl):
            pltpu.async_copy(g_hbm.at[rj[sl]], g_b[p], sem_g[p])
            e0 = pl.multiple_of((rbase + r) * CH, 8)
            pltpu.async_copy(w_hbm.at[tid, pl.ds(e0, CH)], w_b[p], sem_w[p])

        def wait_in(p):
            pltpu.make_async_copy(g_hbm.at[pl.ds(0, CH)], g_b[p], sem_g[p]).wait()
            pltpu.make_async_copy(w_hbm.at[0, pl.ds(0, CH)], w_b[p],
                                  sem_w[p]).wait()

        def wait_scatter(p):
            pltpu.make_async_copy(m_b[p], acc.at[pl.ds(0, CH)], sem_s[p]).wait()

        issue_idx(0, 0)
        issue_idx(1, 1)
        wait_j(0)
        issue_gw(0, 0, 0)

        def quad(q, icarry):
            for jj in range(4):
                r = 4 * q + jj
                p = jj % 2
                sl1 = (jj + 1) % 4
                sl2 = (jj + 2) % 4

                @pl.when(r >= 2)
                def _ws():
                    wait_scatter(p)

                @pl.when(r + 1 < RPT)
                def _next_gw():
                    wait_j(sl1)
                    issue_gw(r + 1, 1 - p, sl1)

                @pl.when(r + 2 < RPT)
                def _next_idx():
                    issue_idx(r + 2, sl2)

                wait_in(p)
                compute(g_b[p], w_b[p], m_b[p])
                wait_i(jj)
                pltpu.async_copy(m_b[p], acc.at[ri[jj]], sem_s[p], add=True)
            return icarry

        lax.fori_loop(0, RPT // 4, quad, 0)
        wait_scatter(0)
        wait_scatter(1)

    def copy_out(tid):
        @pl.when(sid != NS - 1)
        def _copy_full():
            pltpu.sync_copy(acc.at[pl.ds(nbase, NROW)],
                            out_hbm.at[tid, pl.ds(nbase, NROW)])

        @pl.when(sid == NS - 1)
        def _copy_tail():
            pltpu.sync_copy(acc.at[pl.ds(nbase, NROW_LAST)],
                            out_hbm.at[tid, pl.ds(nbase, NROW_LAST)])

    for q in range(NT // NC):
        tid = cid * (NT // NC) + q
        zero_acc()
        plsc.subcore_barrier()
        run_task(tid)
        plsc.subcore_barrier()
        copy_out(tid)


def kernel(s, v, radial_embeddings_1, radial_embeddings_2, f_cut_1, f_cut_2,
           unit_vectors_1, unit_vectors_2, edge_index, W1, b1, W2, b2, Wr, br):
    idx_i = edge_index[0].astype(jnp.int32)
    idx_j = edge_index[1].astype(jnp.int32)
    idxj4 = (idx_j[None, :]
             + (jnp.arange(NT, dtype=jnp.int32) * N)[:, None]).reshape(NT * E)
    fc1 = f_cut_1.reshape(E, 1)
    fc2 = f_cut_2.reshape(E, 1)
    u1 = unit_vectors_1.reshape(E, 1)
    u2 = unit_vectors_2.reshape(E, 1)

    g4 = pl.pallas_call(
        _node_tc_body,
        grid=(NT, N // BN),
        in_specs=[
            pl.BlockSpec((BN, F), lambda t, i: (i, 0)),
            pl.BlockSpec((BN, F), lambda t, i: (i, 0)),
            pl.BlockSpec((F, F), lambda t, i: (0, 0)),
            pl.BlockSpec((1, F), lambda t, i: (0, 0)),
            pl.BlockSpec((F, F), lambda t, i: (0, t)),
            pl.BlockSpec((1, F), lambda t, i: (0, t)),
        ],
        out_specs=pl.BlockSpec((BN, F), lambda t, i: (t * (N // BN) + i, 0)),
        out_shape=jax.ShapeDtypeStruct((NT * N, F), jnp.float32),
    )(s, v, W1.T, b1.reshape(1, F), W2.T, b2.reshape(1, R4F))

    w4 = pl.pallas_call(
        _edge_tc_body,
        grid=(E // BE,),
        in_specs=[
            pl.BlockSpec((BE, R), lambda i: (i, 0)),
            pl.BlockSpec((BE, R), lambda i: (i, 0)),
            pl.BlockSpec((BE, 1), lambda i: (i, 0)),
            pl.BlockSpec((BE, 1), lambda i: (i, 0)),
            pl.BlockSpec((BE, 1), lambda i: (i, 0)),
            pl.BlockSpec((BE, 1), lambda i: (i, 0)),
            pl.BlockSpec((R, R4F), lambda i: (0, 0)),
            pl.BlockSpec((1, R4F), lambda i: (0, 0)),
        ],
        out_specs=pl.BlockSpec((NT, BE, F), lambda i: (0, i, 0)),
        out_shape=jax.ShapeDtypeStruct((NT, E, F), jnp.float32),
    )(radial_embeddings_1, radial_embeddings_2, fc1, fc2, u1, u2,
      Wr.T, br.reshape(1, R4F))

    p4 = _sc_scatter4(g4, w4, idxj4, idx_i)

    out_s, out_v = pl.pallas_call(
        _combine_tc_body,
        grid=(N // BN,),
        in_specs=[
            pl.BlockSpec((BN, F), lambda i: (i, 0)),
            pl.BlockSpec((BN, F), lambda i: (i, 0)),
            pl.BlockSpec((NT, BN, F), lambda i: (0, i, 0)),
        ],
        out_specs=[
            pl.BlockSpec((BN, F), lambda i: (i, 0)),
            pl.BlockSpec((BN, F), lambda i: (i, 0)),
        ],
        out_shape=[
            jax.ShapeDtypeStruct((N, F), jnp.float32),
            jax.ShapeDtypeStruct((N, F), jnp.float32),
        ],
    )(s, v, p4)

    return out_s, out_v


# speedup vs baseline: 52.2778x; 1.8503x over previous
import functools

import jax
import jax.numpy as jnp
from jax import lax
from jax.experimental import pallas as pl
from jax.experimental.pallas import tpu as pltpu
from jax.experimental.pallas import tpu_sc as plsc

N = 10000
E = 320000
F = 128
R = 16
R4F = 512
NT = 4

NC, NS, L = 2, 16, 16
CH = 40
NROWS_ALL = E // CH
RPT = NROWS_ALL // NS
NROW = 640
N_PAD = NROW * NS
ZR = 20
NROW_LAST = N - NROW * (NS - 1)

BN = 2000
BE = 6400
KA = 32


def _node_tc_body(s_ref, v_ref, w1t_ref, b1_ref, w2t_ref, b2_ref, g_ref):
    t = pl.program_id(0)
    h = jnp.dot(s_ref[...], w1t_ref[...], preferred_element_type=jnp.float32)
    h = h + b1_ref[...]
    h = h * jax.nn.sigmoid(h)
    g = jnp.dot(h, w2t_ref[...], preferred_element_type=jnp.float32) + b2_ref[...]
    g_ref[...] = jnp.where(t == 1, g * v_ref[...], g)


def _edge_tc_body(r1t_ref, r2t_ref, fcu_ref, wrt_ref, w_ref):
    fc1 = fcu_ref[0:1, :]
    fc2 = fcu_ref[1:2, :]
    u1 = fcu_ref[2:3, :]
    u2 = fcu_ref[3:4, :]
    a = fc1 * r1t_ref[...] + fc2 * r2t_ref[...]
    lhs = jnp.concatenate(
        [a, fc1 + fc2, jnp.zeros((KA - R - 1, a.shape[1]), jnp.float32)],
        axis=0)
    dn = (((0,), (0,)), ((), ()))
    for t, fac in ((0, None), (1, None), (2, u1), (3, u2)):
        lhs_t = lhs if fac is None else lhs * fac
        w_ref[t] = lax.dot_general(
            lhs_t, wrt_ref[:, t * F:(t + 1) * F], dn,
            preferred_element_type=jnp.float32)


def _combine_tc_body(s_ref, v_ref, p_ref, os_ref, ov_ref):
    os_ref[...] = s_ref[...] + p_ref[0]
    ov_ref[...] = v_ref[...] + (p_ref[1] + p_ref[2] + p_ref[3])


_sc_mesh = plsc.VectorSubcoreMesh(core_axis_name="c", subcore_axis_name="s")

@functools.partial(
    pl.kernel,
    out_type=jax.ShapeDtypeStruct((NT, N, F), jnp.float32),
    mesh=_sc_mesh,
    scratch_types=[
        [pltpu.VMEM((CH,), jnp.int32)] * 4,
        [pltpu.VMEM((CH,), jnp.int32)] * 4,
        [pltpu.VMEM((CH, F), jnp.float32)] * 2,
        [pltpu.VMEM((CH, F), jnp.float32)] * 2,
        [pltpu.VMEM((CH, F), jnp.float32)] * 2,
        pltpu.VMEM((ZR, F), jnp.float32),
        pltpu.VMEM_SHARED((N_PAD, F), jnp.float32),
        [pltpu.SemaphoreType.DMA] * 2,
        [pltpu.SemaphoreType.DMA] * 2,
        [pltpu.SemaphoreType.DMA] * 2,
        [pltpu.SemaphoreType.DMA] * 4,
        [pltpu.SemaphoreType.DMA] * 4,
    ],
)
def _sc_scatter4(g_hbm, w_hbm, idxj_hbm, idxi_hbm, out_hbm,
                 rj, ri, g_b, w_b, m_b, z_v, acc,
                 sem_g, sem_w, sem_s, sem_j, sem_i):
    cid = lax.axis_index("c")
    sid = lax.axis_index("s")
    nbase = pl.multiple_of(sid * NROW, 8)

    zero = jnp.zeros((L,), jnp.float32)

    def zrow(rr, carry):
        for k in range(F // L):
            z_v[rr, pl.ds(k * L, L)] = zero
        return carry

    lax.fori_loop(0, ZR, zrow, 0)

    def zero_acc():
        for q in range(NROW // ZR):
            pltpu.sync_copy(z_v, acc.at[pl.ds(nbase + q * ZR, ZR)])

    def compute(g_v, w_v, m_v):
        def edge(c, icarry):
            for k in range(F // L):
                m_v[c, pl.ds(k * L, L)] = (
                    g_v[c, pl.ds(k * L, L)] * w_v[c, pl.ds(k * L, L)])
            return icarry

        lax.fori_loop(0, CH, edge, 0)

    def run_task(tid):
        rbase = sid * RPT

        def issue_idx(r, sl):
            e0 = pl.multiple_of((rbase + r) * CH, 8)
            pltpu.async_copy(idxj_hbm.at[pl.ds(tid * E + e0, CH)],
                             rj[sl], sem_j[sl])
            pltpu.async_copy(idxi_hbm.at[pl.ds(e0, CH)], ri[sl], sem_i[sl])

        def wait_j(sl):
            pltpu.make_async_copy(idxi_hbm.at[pl.ds(0, CH)], rj[sl],
                                  sem_j[sl]).wait()

        def wait_i(sl):
            pltpu.make_async_copy(idxi_hbm.at[pl.ds(0, CH)], ri[sl],
                                  sem_i[sl]).wait()

        def issue_gw(r, p, sl):
            pltpu.async_copy(g_hbm.at[rj[sl]], g_b[p], sem_g[p])
            e0 = pl.multiple_of((rbase + r) * CH, 8)
            pltpu.async_copy(w_hbm.at[tid, pl.ds(e0, CH)], w_b[p], sem_w[p])

        def wait_in(p):
            pltpu.make_async_copy(g_hbm.at[pl.ds(0, CH)], g_b[p], sem_g[p]).wait()
            pltpu.make_async_copy(w_hbm.at[0, pl.ds(0, CH)], w_b[p],
                                  sem_w[p]).wait()

        def wait_scatter(p):
            pltpu.make_async_copy(m_b[p], acc.at[pl.ds(0, CH)], sem_s[p]).wait()

        issue_idx(0, 0)
        issue_idx(1, 1)
        wait_j(0)
        issue_gw(0, 0, 0)

        def quad(q, icarry):
            for jj in range(4):
                r = 4 * q + jj
                p = jj % 2
                sl1 = (jj + 1) % 4
                sl2 = (jj + 2) % 4

                @pl.when(r >= 2)
                def _ws():
                    wait_scatter(p)

                @pl.when(r + 1 < RPT)
                def _next_gw():
                    wait_j(sl1)
                    issue_gw(r + 1, 1 - p, sl1)

                @pl.when(r + 2 < RPT)
                def _next_idx():
                    issue_idx(r + 2, sl2)

                wait_in(p)
                compute(g_b[p], w_b[p], m_b[p])
                wait_i(jj)
                pltpu.async_copy(m_b[p], acc.at[ri[jj]], sem_s[p], add=True)
            return icarry

        lax.fori_loop(0, RPT // 4, quad, 0)
        wait_scatter(0)
        wait_scatter(1)

    def copy_out(tid):
        @pl.when(sid != NS - 1)
        def _copy_full():
            pltpu.sync_copy(acc.at[pl.ds(nbase, NROW)],
                            out_hbm.at[tid, pl.ds(nbase, NROW)])

        @pl.when(sid == NS - 1)
        def _copy_tail():
            pltpu.sync_copy(acc.at[pl.ds(nbase, NROW_LAST)],
                            out_hbm.at[tid, pl.ds(nbase, NROW_LAST)])

    for q in range(NT // NC):
        tid = cid * (NT // NC) + q
        zero_acc()
        plsc.subcore_barrier()
        run_task(tid)
        plsc.subcore_barrier()
        copy_out(tid)


def kernel(s, v, radial_embeddings_1, radial_embeddings_2, f_cut_1, f_cut_2,
           unit_vectors_1, unit_vectors_2, edge_index, W1, b1, W2, b2, Wr, br):
    idx_i = edge_index[0].astype(jnp.int32)
    idx_j = edge_index[1].astype(jnp.int32)
    idxj4 = (idx_j[None, :]
             + (jnp.arange(NT, dtype=jnp.int32) * N)[:, None]).reshape(NT * E)
    r1t = radial_embeddings_1.T
    r2t = radial_embeddings_2.T
    fcu = jnp.concatenate(
        [jnp.stack([f_cut_1, f_cut_2, unit_vectors_1, unit_vectors_2]),
         jnp.zeros((4, E), jnp.float32)], axis=0)
    wrt_aug = jnp.concatenate(
        [Wr.T, br[None, :], jnp.zeros((KA - R - 1, R4F), jnp.float32)],
        axis=0)

    g4 = pl.pallas_call(
        _node_tc_body,
        grid=(NT, N // BN),
        in_specs=[
            pl.BlockSpec((BN, F), lambda t, i: (i, 0)),
            pl.BlockSpec((BN, F), lambda t, i: (i, 0)),
            pl.BlockSpec((F, F), lambda t, i: (0, 0)),
            pl.BlockSpec((1, F), lambda t, i: (0, 0)),
            pl.BlockSpec((F, F), lambda t, i: (0, t)),
            pl.BlockSpec((1, F), lambda t, i: (0, t)),
        ],
        out_specs=pl.BlockSpec((BN, F), lambda t, i: (t * (N // BN) + i, 0)),
        out_shape=jax.ShapeDtypeStruct((NT * N, F), jnp.float32),
    )(s, v, W1.T, b1.reshape(1, F), W2.T, b2.reshape(1, R4F))

    w4 = pl.pallas_call(
        _edge_tc_body,
        grid=(E // BE,),
        in_specs=[
            pl.BlockSpec((R, BE), lambda i: (0, i)),
            pl.BlockSpec((R, BE), lambda i: (0, i)),
            pl.BlockSpec((8, BE), lambda i: (0, i)),
            pl.BlockSpec((KA, R4F), lambda i: (0, 0)),
        ],
        out_specs=pl.BlockSpec((NT, BE, F), lambda i: (0, i, 0)),
        out_shape=jax.ShapeDtypeStruct((NT, E, F), jnp.float32),
    )(r1t, r2t, fcu, wrt_aug)

    p4 = _sc_scatter4(g4, w4, idxj4, idx_i)

    out_s, out_v = pl.pallas_call(
        _combine_tc_body,
        grid=(N // BN,),
        in_specs=[
            pl.BlockSpec((BN, F), lambda i: (i, 0)),
            pl.BlockSpec((BN, F), lambda i: (i, 0)),
            pl.BlockSpec((NT, BN, F), lambda i: (0, i, 0)),
        ],
        out_specs=[
            pl.BlockSpec((BN, F), lambda i: (i, 0)),
            pl.BlockSpec((BN, F), lambda i: (i, 0)),
        ],
        out_shape=[
            jax.ShapeDtypeStruct((N, F), jnp.float32),
            jax.ShapeDtypeStruct((N, F), jnp.float32),
        ],
    )(s, v, p4)

    return out_s, out_v
